# baseline (speedup 1.0000x reference)
import jax
import jax.numpy as jnp
from jax import lax
from jax.experimental import pallas as pl
from jax.experimental.pallas import tpu as pltpu
from jax.experimental.pallas import tpu_sc as plsc

BATCH = 16384
EMBED_DIM = 64
PAIR = 128
NUM_NODES = 1000000
NUM_RELS = 1000
NUM_CORES = 2
NUM_SUBCORES = 16
NUM_WORKERS = NUM_CORES * NUM_SUBCORES
B_PER_W = BATCH // NUM_WORKERS
CHUNK = 128
CHUNKS_PER_W = B_PER_W // CHUNK
LANES = 16
GROUPS_PER_CHUNK = CHUNK // LANES


def _body(node_hbm, rel_hbm, head_hbm, relids_hbm, tail_hbm,
          hcol_hbm, rcol_hbm, tcol_hbm, out_hbm,
          idx_h, idx_r, idx_t, col_h_v, col_r_v, col_t_v,
          h_buf, r_buf, t_buf, o_buf,
          sem_h, sem_r, sem_t):
    wid = lax.axis_index("s") * NUM_CORES + lax.axis_index("c")
    base = wid * B_PER_W
    idx_row = wid * CHUNKS_PER_W

    pltpu.sync_copy(head_hbm.at[pl.ds(idx_row, CHUNKS_PER_W)], idx_h)
    pltpu.sync_copy(relids_hbm.at[pl.ds(idx_row, CHUNKS_PER_W)], idx_r)
    pltpu.sync_copy(tail_hbm.at[pl.ds(idx_row, CHUNKS_PER_W)], idx_t)
    pltpu.sync_copy(hcol_hbm.at[pl.ds(idx_row, CHUNKS_PER_W)], col_h_v)
    pltpu.sync_copy(rcol_hbm.at[pl.ds(idx_row, CHUNKS_PER_W)], col_r_v)
    pltpu.sync_copy(tcol_hbm.at[pl.ds(idx_row, CHUNKS_PER_W)], col_t_v)

    iota = lax.broadcasted_iota(jnp.int32, (LANES,), 0)

    for c in range(CHUNKS_PER_W):
        ch = pltpu.async_copy(node_hbm.at[idx_h.at[c]], h_buf, sem_h)
        cr = pltpu.async_copy(rel_hbm.at[idx_r.at[c]], r_buf, sem_r)
        ct = pltpu.async_copy(node_hbm.at[idx_t.at[c]], t_buf, sem_t)
        ch.wait()
        cr.wait()
        ct.wait()

        def group_body(g, carry):
            slot = g * LANES + iota
            col_h = col_h_v[c, pl.ds(g * LANES, LANES)]
            col_r = col_r_v[c, pl.ds(g * LANES, LANES)]
            col_t = col_t_v[c, pl.ds(g * LANES, LANES)]
            for k in range(EMBED_DIM):
                h = plsc.load_gather(h_buf, [slot, col_h + k])
                r = plsc.load_gather(r_buf, [slot, col_r + k])
                t = plsc.load_gather(t_buf, [slot, col_t + k])
                o_buf[k, pl.ds(c * CHUNK + g * LANES, LANES)] = h + r + t
            return carry

        lax.fori_loop(0, GROUPS_PER_CHUNK, group_body, 0)

    pltpu.sync_copy(o_buf, out_hbm.at[:, pl.ds(base, B_PER_W)])


@jax.jit
def kernel(head_ids, rel_ids, tail_ids, node_embeddings, rel_weight):
    mesh = plsc.VectorSubcoreMesh(core_axis_name="c", subcore_axis_name="s",
                                  num_cores=NUM_CORES,
                                  num_subcores=NUM_SUBCORES)
    k = pl.kernel(
        _body,
        out_type=jax.ShapeDtypeStruct((EMBED_DIM, BATCH), jnp.float32),
        mesh=mesh,
        compiler_params=pltpu.CompilerParams(needs_layout_passes=False),
        scratch_types=[
            pltpu.VMEM((CHUNKS_PER_W, CHUNK), jnp.int32),
            pltpu.VMEM((CHUNKS_PER_W, CHUNK), jnp.int32),
            pltpu.VMEM((CHUNKS_PER_W, CHUNK), jnp.int32),
            pltpu.VMEM((CHUNKS_PER_W, CHUNK), jnp.int32),
            pltpu.VMEM((CHUNKS_PER_W, CHUNK), jnp.int32),
            pltpu.VMEM((CHUNKS_PER_W, CHUNK), jnp.int32),
            pltpu.VMEM((CHUNK, PAIR), jnp.float32),
            pltpu.VMEM((CHUNK, PAIR), jnp.float32),
            pltpu.VMEM((CHUNK, PAIR), jnp.float32),
            pltpu.VMEM((EMBED_DIM, B_PER_W), jnp.float32),
            pltpu.SemaphoreType.DMA,
            pltpu.SemaphoreType.DMA,
            pltpu.SemaphoreType.DMA,
        ],
    )
    node_pairs = node_embeddings.reshape(NUM_NODES // 2, PAIR)
    rel_pairs = rel_weight.reshape(NUM_RELS // 2, PAIR)
    nrows = NUM_WORKERS * CHUNKS_PER_W
    head2d = (head_ids >> 1).reshape(nrows, CHUNK)
    rel2d = (rel_ids >> 1).reshape(nrows, CHUNK)
    tail2d = (tail_ids >> 1).reshape(nrows, CHUNK)
    hcol = ((head_ids & 1) * EMBED_DIM).reshape(nrows, CHUNK)
    rcol = ((rel_ids & 1) * EMBED_DIM).reshape(nrows, CHUNK)
    tcol = ((tail_ids & 1) * EMBED_DIM).reshape(nrows, CHUNK)
    out_t = k(node_pairs, rel_pairs, head2d, rel2d, tail2d, hcol, rcol, tcol)
    return out_t.T
